# Initial kernel scaffold; baseline (speedup 1.0000x reference)
import jax
import jax.numpy as jnp
from jax.experimental import pallas as pl


def kernel(input_token, pos_token, segment_token, W_in, W_seg, W_pos, gamma, beta):
    raise NotImplementedError("write your pallas kernel here")


# speedup vs baseline: 3.7528x; 3.7528x over previous
import functools

import jax
import jax.numpy as jnp
from jax import lax
from jax.experimental import pallas as pl
from jax.experimental.pallas import tpu as pltpu
from jax.experimental.pallas import tpu_sc as plsc

D_MODEL = 768
VOCAB = 4
MAX_LEN = 30
N_SEG = 2
N_COMB = VOCAB * MAX_LEN * N_SEG

NC = 2
NS = 16
NW = NC * NS

B_TOT = 4096 * 30
BPW = B_TOT // NW
CHUNK = 64
NCHUNK = BPW // CHUNK


def _table_body(win_ref, wpos_ref, wseg_ref, g_ref, b_ref, out_ref):
    rid = lax.broadcasted_iota(jnp.int32, (N_COMB, 1), 0)
    v = rid // (MAX_LEN * N_SEG)
    p = (rid // N_SEG) % MAX_LEN
    g = rid % N_SEG
    oh_v = (v == lax.broadcasted_iota(jnp.int32, (N_COMB, VOCAB), 1)).astype(jnp.float32)
    oh_p = (p == lax.broadcasted_iota(jnp.int32, (N_COMB, MAX_LEN), 1)).astype(jnp.float32)
    oh_g = (g == lax.broadcasted_iota(jnp.int32, (N_COMB, N_SEG), 1)).astype(jnp.float32)
    f = (jnp.dot(oh_v, win_ref[...], preferred_element_type=jnp.float32)
         + jnp.dot(oh_p, wpos_ref[...], preferred_element_type=jnp.float32)
         + jnp.dot(oh_g, wseg_ref[...], preferred_element_type=jnp.float32))
    mean = jnp.mean(f, axis=1, keepdims=True)
    d = f - mean
    var = jnp.mean(d * d, axis=1, keepdims=True)
    out_ref[...] = (d * lax.rsqrt(var + 1e-5)) * g_ref[...] + b_ref[...]


def _build_table(w_in, w_pos, w_seg, gamma, beta):
    return pl.pallas_call(
        _table_body,
        out_shape=jax.ShapeDtypeStruct((N_COMB, D_MODEL), jnp.float32),
    )(w_in, w_pos, w_seg, gamma.reshape(1, D_MODEL), beta.reshape(1, D_MODEL))


@functools.cache
def _make_gather_kernel():
    @functools.partial(
        pl.kernel,
        out_type=jax.ShapeDtypeStruct((B_TOT, D_MODEL), jnp.float32),
        mesh=plsc.VectorSubcoreMesh(core_axis_name="c", subcore_axis_name="s"),
        scratch_types=[
            pltpu.VMEM((BPW,), jnp.int32),
            pltpu.VMEM((BPW,), jnp.int32),
            pltpu.VMEM((BPW,), jnp.int32),
            pltpu.VMEM((NCHUNK, CHUNK), jnp.int32),
            pltpu.VMEM((CHUNK, D_MODEL), jnp.float32),
            pltpu.SemaphoreType.DMA,
        ],
    )
    def _gather_kernel(table, it_hbm, pt_hbm, st_hbm, out, it_v, pt_v, st_v,
                       idx_v, rows_v, gsem):
        c = lax.axis_index("c")
        s = lax.axis_index("s")
        wid = s * NC + c
        base = wid * BPW
        pltpu.sync_copy(it_hbm.at[pl.ds(base, BPW)], it_v)
        pltpu.sync_copy(pt_hbm.at[pl.ds(base, BPW)], pt_v)
        pltpu.sync_copy(st_hbm.at[pl.ds(base, BPW)], st_v)

        def idx_body(ci, carry):
            for j in range(CHUNK // 16):
                sl = pl.ds(ci * CHUNK + j * 16, 16)
                r = (it_v[sl] * (MAX_LEN * N_SEG) + pt_v[sl] * N_SEG + st_v[sl])
                idx_v[ci, pl.ds(j * 16, 16)] = r
            return carry

        lax.fori_loop(0, NCHUNK, idx_body, 0)

        def chunk_body(ci, carry):
            pltpu.async_copy(table.at[idx_v.at[ci]], rows_v, gsem).wait()
            pltpu.sync_copy(rows_v, out.at[pl.ds(base + ci * CHUNK, CHUNK)])
            return carry

        lax.fori_loop(0, NCHUNK, chunk_body, 0)

    return _gather_kernel


def kernel(input_token, pos_token, segment_token, W_in, W_seg, W_pos, gamma, beta):
    table = _build_table(W_in, W_pos, W_seg, gamma, beta)
    it = input_token.reshape(-1).astype(jnp.int32)
    pt = pos_token.reshape(-1).astype(jnp.int32)
    st = segment_token.reshape(-1).astype(jnp.int32)
    out = _make_gather_kernel()(table, it, pt, st)
    return out.reshape(input_token.shape[0], input_token.shape[1], D_MODEL)


# speedup vs baseline: 3.8202x; 1.0180x over previous
import functools

import jax
import jax.numpy as jnp
from jax import lax
from jax.experimental import pallas as pl
from jax.experimental.pallas import tpu as pltpu
from jax.experimental.pallas import tpu_sc as plsc

D_MODEL = 768
VOCAB = 4
MAX_LEN = 30
N_SEG = 2
N_COMB = VOCAB * MAX_LEN * N_SEG

NC = 2
NS = 16
NW = NC * NS

B_TOT = 4096 * 30
BPW = B_TOT // NW
CHUNK = 64
NCHUNK = BPW // CHUNK


def _table_body(win_ref, wpos_ref, wseg_ref, g_ref, b_ref, out_ref):
    rid = lax.broadcasted_iota(jnp.int32, (N_COMB, 1), 0)
    v = rid // (MAX_LEN * N_SEG)
    p = (rid // N_SEG) % MAX_LEN
    g = rid % N_SEG
    oh_v = (v == lax.broadcasted_iota(jnp.int32, (N_COMB, VOCAB), 1)).astype(jnp.float32)
    oh_p = (p == lax.broadcasted_iota(jnp.int32, (N_COMB, MAX_LEN), 1)).astype(jnp.float32)
    oh_g = (g == lax.broadcasted_iota(jnp.int32, (N_COMB, N_SEG), 1)).astype(jnp.float32)
    f = (jnp.dot(oh_v, win_ref[...], preferred_element_type=jnp.float32)
         + jnp.dot(oh_p, wpos_ref[...], preferred_element_type=jnp.float32)
         + jnp.dot(oh_g, wseg_ref[...], preferred_element_type=jnp.float32))
    mean = jnp.mean(f, axis=1, keepdims=True)
    d = f - mean
    var = jnp.mean(d * d, axis=1, keepdims=True)
    out_ref[...] = (d * lax.rsqrt(var + 1e-5)) * g_ref[...] + b_ref[...]


def _build_table(w_in, w_pos, w_seg, gamma, beta):
    return pl.pallas_call(
        _table_body,
        out_shape=jax.ShapeDtypeStruct((N_COMB, D_MODEL), jnp.float32),
    )(w_in, w_pos, w_seg, gamma.reshape(1, D_MODEL), beta.reshape(1, D_MODEL))


@functools.cache
def _make_gather_kernel():
    @functools.partial(
        pl.kernel,
        out_type=jax.ShapeDtypeStruct((B_TOT, D_MODEL), jnp.float32),
        mesh=plsc.VectorSubcoreMesh(core_axis_name="c", subcore_axis_name="s"),
        scratch_types=[
            pltpu.VMEM((BPW,), jnp.int32),
            pltpu.VMEM((BPW,), jnp.int32),
            pltpu.VMEM((BPW,), jnp.int32),
            pltpu.VMEM((NCHUNK, CHUNK), jnp.int32),
            pltpu.VMEM((2, CHUNK, D_MODEL), jnp.float32),
            pltpu.SemaphoreType.DMA,
            pltpu.SemaphoreType.DMA,
        ],
    )
    def _gather_kernel(table, it_hbm, pt_hbm, st_hbm, out, it_v, pt_v, st_v,
                       idx_v, rows_v, gsem, wsem):
        c = lax.axis_index("c")
        s = lax.axis_index("s")
        wid = s * NC + c
        base = wid * BPW
        pltpu.sync_copy(it_hbm.at[pl.ds(base, BPW)], it_v)
        pltpu.sync_copy(pt_hbm.at[pl.ds(base, BPW)], pt_v)
        pltpu.sync_copy(st_hbm.at[pl.ds(base, BPW)], st_v)

        def idx_body(ci, carry):
            for j in range(CHUNK // 16):
                sl = pl.ds(ci * CHUNK + j * 16, 16)
                r = (it_v[sl] * (MAX_LEN * N_SEG) + pt_v[sl] * N_SEG + st_v[sl])
                idx_v[ci, pl.ds(j * 16, 16)] = r
            return carry

        lax.fori_loop(0, NCHUNK, idx_body, 0)

        def start_gather(ci, b):
            pltpu.async_copy(table.at[idx_v.at[ci]], rows_v.at[b], gsem)

        def wait_gather(b):
            pltpu.make_async_copy(table.at[pl.ds(0, CHUNK)], rows_v.at[b], gsem).wait()

        def start_write(ci, b):
            pltpu.async_copy(rows_v.at[b], out.at[pl.ds(base + ci * CHUNK, CHUNK)], wsem)

        def wait_write(b):
            pltpu.make_async_copy(rows_v.at[b], out.at[pl.ds(base, CHUNK)], wsem).wait()

        start_gather(0, 0)
        start_gather(1, 1)

        def pair_body(cp, carry):
            for b in range(2):
                ci = cp * 2 + b
                wait_gather(b)
                start_write(ci, b)
                wait_write(b)
                start_gather(ci + 2, b)
            return carry

        lax.fori_loop(0, NCHUNK // 2 - 1, pair_body, 0)
        for b in range(2):
            wait_gather(b)
            start_write(NCHUNK - 2 + b, b)
        for b in range(2):
            wait_write(b)

    return _gather_kernel


def kernel(input_token, pos_token, segment_token, W_in, W_seg, W_pos, gamma, beta):
    table = _build_table(W_in, W_pos, W_seg, gamma, beta)
    it = input_token.reshape(-1).astype(jnp.int32)
    pt = pos_token.reshape(-1).astype(jnp.int32)
    st = segment_token.reshape(-1).astype(jnp.int32)
    out = _make_gather_kernel()(table, it, pt, st)
    return out.reshape(input_token.shape[0], input_token.shape[1], D_MODEL)


# speedup vs baseline: 3.8245x; 1.0011x over previous
import functools

import jax
import jax.numpy as jnp
from jax import lax
from jax.experimental import pallas as pl
from jax.experimental.pallas import tpu as pltpu
from jax.experimental.pallas import tpu_sc as plsc

D_MODEL = 768
VOCAB = 4
MAX_LEN = 30
N_SEG = 2
N_COMB = VOCAB * MAX_LEN * N_SEG

NC = 2
NS = 16
NW = NC * NS

B_TOT = 4096 * 30
BPW = B_TOT // NW
CHUNK = 32
NCHUNK = BPW // CHUNK
SROWS = NS * CHUNK
NSTEP = NCHUNK
ROWS_PER_SC = B_TOT // NC


def _table_body(win_ref, wpos_ref, wseg_ref, g_ref, b_ref, out_ref):
    rid = lax.broadcasted_iota(jnp.int32, (N_COMB, 1), 0)
    v = rid // (MAX_LEN * N_SEG)
    p = (rid // N_SEG) % MAX_LEN
    g = rid % N_SEG
    oh_v = (v == lax.broadcasted_iota(jnp.int32, (N_COMB, VOCAB), 1)).astype(jnp.float32)
    oh_p = (p == lax.broadcasted_iota(jnp.int32, (N_COMB, MAX_LEN), 1)).astype(jnp.float32)
    oh_g = (g == lax.broadcasted_iota(jnp.int32, (N_COMB, N_SEG), 1)).astype(jnp.float32)
    f = (jnp.dot(oh_v, win_ref[...], preferred_element_type=jnp.float32)
         + jnp.dot(oh_p, wpos_ref[...], preferred_element_type=jnp.float32)
         + jnp.dot(oh_g, wseg_ref[...], preferred_element_type=jnp.float32))
    mean = jnp.mean(f, axis=1, keepdims=True)
    d = f - mean
    var = jnp.mean(d * d, axis=1, keepdims=True)
    out_ref[...] = (d * lax.rsqrt(var + 1e-5)) * g_ref[...] + b_ref[...]


def _build_table(w_in, w_pos, w_seg, gamma, beta):
    return pl.pallas_call(
        _table_body,
        out_shape=jax.ShapeDtypeStruct((N_COMB, D_MODEL), jnp.float32),
    )(w_in, w_pos, w_seg, gamma.reshape(1, D_MODEL), beta.reshape(1, D_MODEL))


@functools.cache
def _make_gather_kernel():
    @functools.partial(
        pl.kernel,
        out_type=jax.ShapeDtypeStruct((B_TOT, D_MODEL), jnp.float32),
        mesh=plsc.VectorSubcoreMesh(core_axis_name="c", subcore_axis_name="s"),
        scratch_types=[
            pltpu.VMEM((BPW,), jnp.int32),
            pltpu.VMEM((BPW,), jnp.int32),
            pltpu.VMEM((BPW,), jnp.int32),
            pltpu.VMEM((NCHUNK, CHUNK), jnp.int32),
            pltpu.VMEM((2, CHUNK, D_MODEL), jnp.float32),
            pltpu.VMEM_SHARED((2, SROWS, D_MODEL), jnp.float32),
            pltpu.SemaphoreType.DMA,
            pltpu.SemaphoreType.DMA,
        ],
    )
    def _gather_kernel(table, it_hbm, pt_hbm, st_hbm, out, it_v, pt_v, st_v,
                       idx_v, rows_v, sp_buf, gsem, wsem):
        c = lax.axis_index("c")
        s = lax.axis_index("s")
        base_tok = (c * NS + s) * BPW
        pltpu.sync_copy(it_hbm.at[pl.ds(base_tok, BPW)], it_v)
        pltpu.sync_copy(pt_hbm.at[pl.ds(base_tok, BPW)], pt_v)
        pltpu.sync_copy(st_hbm.at[pl.ds(base_tok, BPW)], st_v)

        def idx_body(ci, carry):
            for j in range(CHUNK // 16):
                sl = pl.ds(ci * CHUNK + j * 16, 16)
                r = (it_v[sl] * (MAX_LEN * N_SEG) + pt_v[sl] * N_SEG + st_v[sl])
                idx_v[ci, pl.ds(j * 16, 16)] = r
            return carry

        lax.fori_loop(0, NCHUNK, idx_body, 0)

        def start_gather(ci, b):
            pltpu.async_copy(table.at[idx_v.at[ci]], rows_v.at[b], gsem)

        def wait_gather(b):
            pltpu.make_async_copy(table.at[pl.ds(0, CHUNK)], rows_v.at[b], gsem).wait()

        def start_write(ci, b):
            pltpu.async_copy(sp_buf.at[b],
                             out.at[pl.ds(c * ROWS_PER_SC + ci * SROWS, SROWS)],
                             wsem)

        def wait_write():
            pltpu.make_async_copy(sp_buf.at[0], out.at[pl.ds(0, SROWS)], wsem).wait()

        start_gather(0, 0)
        start_gather(1, 1)

        def pair_body(cp, carry):
            for b in range(2):
                ci = cp * 2 + b

                @pl.when(jnp.logical_and(s == 0, cp > 0))
                def _wait_prev_write():
                    wait_write()

                plsc.subcore_barrier()
                wait_gather(b)
                pltpu.sync_copy(rows_v.at[b], sp_buf.at[b, pl.ds(s * CHUNK, CHUNK)])
                plsc.subcore_barrier()

                @pl.when(s == 0)
                def _fire_write():
                    start_write(ci, b)

                @pl.when(ci + 2 < NSTEP)
                def _next_gather():
                    start_gather(ci + 2, b)

            return carry

        lax.fori_loop(0, NSTEP // 2, pair_body, 0)

        @pl.when(s == 0)
        def _drain():
            wait_write()
            wait_write()

    return _gather_kernel


def kernel(input_token, pos_token, segment_token, W_in, W_seg, W_pos, gamma, beta):
    table = _build_table(W_in, W_pos, W_seg, gamma, beta)

    def permute(a):
        return (a.reshape(-1).astype(jnp.int32)
                 .reshape(NC, NSTEP, NS, CHUNK)
                 .transpose(0, 2, 1, 3)
                 .reshape(-1))

    out = _make_gather_kernel()(table, permute(input_token),
                                permute(pos_token), permute(segment_token))
    return out.reshape(input_token.shape[0], input_token.shape[1], D_MODEL)


# speedup vs baseline: 3.9618x; 1.0359x over previous
import functools

import jax
import jax.numpy as jnp
from jax import lax
from jax.experimental import pallas as pl
from jax.experimental.pallas import tpu as pltpu
from jax.experimental.pallas import tpu_sc as plsc

D_MODEL = 768
VOCAB = 4
MAX_LEN = 30
N_SEG = 2
N_COMB = VOCAB * MAX_LEN * N_SEG

NC = 2
NS = 16
NW = NC * NS

B_TOT = 4096 * 30
CHUNK = 32
NBUF = 4
SC_ROWS = 61440
TC_BLK = 512


def _table_body(win_ref, wpos_ref, wseg_ref, g_ref, b_ref, out_ref):
    rid = lax.broadcasted_iota(jnp.int32, (N_COMB, 1), 0)
    v = rid // (MAX_LEN * N_SEG)
    p = (rid // N_SEG) % MAX_LEN
    g = rid % N_SEG
    oh_v = (v == lax.broadcasted_iota(jnp.int32, (N_COMB, VOCAB), 1)).astype(jnp.float32)
    oh_p = (p == lax.broadcasted_iota(jnp.int32, (N_COMB, MAX_LEN), 1)).astype(jnp.float32)
    oh_g = (g == lax.broadcasted_iota(jnp.int32, (N_COMB, N_SEG), 1)).astype(jnp.float32)
    f = (jnp.dot(oh_v, win_ref[...], preferred_element_type=jnp.float32)
         + jnp.dot(oh_p, wpos_ref[...], preferred_element_type=jnp.float32)
         + jnp.dot(oh_g, wseg_ref[...], preferred_element_type=jnp.float32))
    mean = jnp.mean(f, axis=1, keepdims=True)
    d = f - mean
    var = jnp.mean(d * d, axis=1, keepdims=True)
    out_ref[...] = (d * lax.rsqrt(var + 1e-5)) * g_ref[...] + b_ref[...]


def _build_table(w_in, w_pos, w_seg, gamma, beta):
    return pl.pallas_call(
        _table_body,
        out_shape=jax.ShapeDtypeStruct((N_COMB, D_MODEL), jnp.float32),
    )(w_in, w_pos, w_seg, gamma.reshape(1, D_MODEL), beta.reshape(1, D_MODEL))


@functools.cache
def _make_gather_kernel(sc_rows):
    BPW = sc_rows // NW
    NCHUNK = BPW // CHUNK

    @functools.partial(
        pl.kernel,
        out_type=jax.ShapeDtypeStruct((B_TOT, D_MODEL), jnp.float32),
        mesh=plsc.VectorSubcoreMesh(core_axis_name="c", subcore_axis_name="s"),
        scratch_types=[
            pltpu.VMEM((BPW,), jnp.int32),
            pltpu.VMEM((BPW,), jnp.int32),
            pltpu.VMEM((BPW,), jnp.int32),
            pltpu.VMEM((NCHUNK, CHUNK), jnp.int32),
            pltpu.VMEM((NBUF, CHUNK, D_MODEL), jnp.float32),
            pltpu.SemaphoreType.DMA,
            pltpu.SemaphoreType.DMA,
        ],
    )
    def _gather_kernel(table, it_hbm, pt_hbm, st_hbm, out, it_v, pt_v, st_v,
                       idx_v, rows_v, gsem, wsem):
        c = lax.axis_index("c")
        s = lax.axis_index("s")
        wid = s * NC + c
        base = wid * BPW
        pltpu.sync_copy(it_hbm.at[pl.ds(base, BPW)], it_v)
        pltpu.sync_copy(pt_hbm.at[pl.ds(base, BPW)], pt_v)
        pltpu.sync_copy(st_hbm.at[pl.ds(base, BPW)], st_v)

        def idx_body(ci, carry):
            for j in range(CHUNK // 16):
                sl = pl.ds(ci * CHUNK + j * 16, 16)
                r = (it_v[sl] * (MAX_LEN * N_SEG) + pt_v[sl] * N_SEG + st_v[sl])
                idx_v[ci, pl.ds(j * 16, 16)] = r
            return carry

        lax.fori_loop(0, NCHUNK, idx_body, 0)

        def start_gather(ci, b):
            pltpu.async_copy(table.at[idx_v.at[ci]], rows_v.at[b], gsem)

        def wait_gather(b):
            pltpu.make_async_copy(table.at[pl.ds(0, CHUNK)], rows_v.at[b], gsem).wait()

        def start_write(ci, b):
            pltpu.async_copy(rows_v.at[b], out.at[pl.ds(base + ci * CHUNK, CHUNK)], wsem)

        def wait_write(b):
            pltpu.make_async_copy(rows_v.at[b], out.at[pl.ds(base, CHUNK)], wsem).wait()

        for b in range(NBUF):
            start_gather(b, b)

        def grp_body(cp, carry):
            for b in range(NBUF):
                ci = cp * NBUF + b
                wait_gather(b)
                start_write(ci, b)

                @pl.when(ci + NBUF < NCHUNK)
                def _refill():
                    wait_write(b)
                    start_gather(ci + NBUF, b)

            return carry

        lax.fori_loop(0, NCHUNK // NBUF, grp_body, 0)
        for b in range(NBUF):
            wait_write(b)

    return _gather_kernel


def _tc_fill_body(buf_ref, it_ref, pt_ref, st_ref, tab_ref, out_ref):
    del buf_ref
    r = (it_ref[...] * (MAX_LEN * N_SEG) + pt_ref[...] * N_SEG
         + st_ref[...]).reshape(1, TC_BLK)
    oh = (lax.broadcasted_iota(jnp.int32, (N_COMB, TC_BLK), 0) == r
          ).astype(jnp.bfloat16)
    out_ref[...] = lax.dot_general(oh, tab_ref[...], (((0,), (0,)), ((), ())),
                                   preferred_element_type=jnp.float32)


def _tc_fill(sc_out, it, pt, st, tab16):
    g = (B_TOT - SC_ROWS) // TC_BLK
    off = SC_ROWS // TC_BLK

    def tok3(a):
        return a[SC_ROWS:].reshape(g, 1, TC_BLK)

    tok_spec = pl.BlockSpec((1, 1, TC_BLK), lambda i: (i, 0, 0))
    return pl.pallas_call(
        _tc_fill_body,
        grid=(g,),
        in_specs=[
            pl.BlockSpec(memory_space=pl.ANY),
            tok_spec, tok_spec, tok_spec,
            pl.BlockSpec((N_COMB, D_MODEL), lambda i: (0, 0)),
        ],
        out_specs=pl.BlockSpec((TC_BLK, D_MODEL), lambda i: (off + i, 0)),
        out_shape=jax.ShapeDtypeStruct((B_TOT, D_MODEL), jnp.float32),
        input_output_aliases={0: 0},
    )(sc_out, tok3(it), tok3(pt), tok3(st), tab16)


def kernel(input_token, pos_token, segment_token, W_in, W_seg, W_pos, gamma, beta):
    table = _build_table(W_in, W_pos, W_seg, gamma, beta)
    it = input_token.reshape(-1).astype(jnp.int32)
    pt = pos_token.reshape(-1).astype(jnp.int32)
    st = segment_token.reshape(-1).astype(jnp.int32)
    sc_out = _make_gather_kernel(SC_ROWS)(table, it, pt, st)
    out = _tc_fill(sc_out, it, pt, st, table.astype(jnp.bfloat16))
    return out.reshape(input_token.shape[0], input_token.shape[1], D_MODEL)
